# baseline (speedup 1.0000x reference)
import jax
import jax.numpy as jnp
from jax.experimental import pallas as pl

K = 2
TAU = 0.35
ALPHA_SELFLOOP = 0.5
EPS = 1e-06
LAMBDA_FUSE = 0.05

_M = 10
_CV = 100


def _taglayer_body(x_ref, lam_ref, o_ref):
    B = x_ref.shape[-1]
    f32 = jnp.float32

    cen = []
    mag = []
    cen_d = [[], [], []]
    for m in range(_M):
        g = x_ref[m][0:75, :].astype(f32)
        mag.append(jnp.sum(jnp.abs(g), axis=0))
        for d in range(3):
            cen_d[d].append(jnp.mean(g[d * 25:(d + 1) * 25, :], axis=0))
    presf = (jnp.stack(mag, axis=0) > EPS).astype(f32)

    dsq = jnp.zeros((_M, _M, B), dtype=f32)
    for d in range(3):
        cd = jnp.stack(cen_d[d], axis=0)
        diff = cd[:, None, :] - cd[None, :, :]
        dsq = dsq + diff * diff
    dist = jnp.sqrt(jnp.clip(dsq, 1e-12, None))
    pair_ok = presf[:, None, :] * presf[None, :, :] > 0
    dist = jnp.where(pair_ok, dist, 1000000.0)
    ii = jax.lax.broadcasted_iota(jnp.int32, (_M, _M, B), 0)
    jj = jax.lax.broadcasted_iota(jnp.int32, (_M, _M, B), 1)
    eye3 = (ii == jj).astype(f32)
    dist = dist + eye3 * 1000000.0

    d1 = dist[0]
    i1 = jnp.zeros((_M, B), dtype=jnp.int32)
    for j in range(1, _M):
        dj = dist[j]
        take = dj < d1
        d1 = jnp.where(take, dj, d1)
        i1 = jnp.where(take, j, i1)
    big = jnp.float32(3.0e38)
    d2 = jnp.where(i1 == 0, big, dist[0])
    i2 = jnp.zeros((_M, B), dtype=jnp.int32)
    for j in range(1, _M):
        dj = dist[j]
        take = jnp.logical_and(i1 != j, dj < d2)
        d2 = jnp.where(take, dj, d2)
        i2 = jnp.where(take, j, i2)

    w1 = jnp.exp(-d1 / TAU)
    w2 = jnp.exp(-d2 / TAU)
    s = w1 + w2 + 1e-06
    w1 = w1 / s
    w2 = w2 / s
    cols = []
    for j in range(_M):
        cols.append(w1 * (i1 == j).astype(f32) + w2 * (i2 == j).astype(f32))
    adjT = jnp.stack(cols, axis=0)

    adjT = 0.5 * (adjT + jnp.transpose(adjT, (1, 0, 2)))
    adjT = adjT + ALPHA_SELFLOOP * eye3
    rs = jnp.sum(adjT, axis=0)
    adjT = adjT / (rs + 1e-06)[None, :, :]
    deg = jnp.clip(jnp.sum(adjT, axis=0), 1e-06, None)
    dinv = jax.lax.rsqrt(deg)
    dinv_j = dinv[:, None, :]
    adjT = adjT * dinv[None, :, :] * dinv_j

    lamb = lam_ref[0, 0].astype(jnp.bfloat16)
    adjb = adjT.astype(jnp.bfloat16)
    for i in range(_M):
        acc = None
        for j in range(_M):
            t = adjb[j, i:i + 1, :] * x_ref[j]
            acc = t if acc is None else acc + t
        o_ref[i] = x_ref[i] + lamb * acc


def kernel(x, lam):
    N, C, T, V, M = x.shape
    NT = N * T
    B = 512
    xt = jnp.transpose(x.astype(jnp.bfloat16), (4, 1, 3, 0, 2))
    xt = xt.reshape(M, C * V, NT)
    lam2 = jnp.asarray(lam, jnp.float32).reshape(1, 1)
    out3 = pl.pallas_call(
        _taglayer_body,
        grid=(NT // B,),
        in_specs=[
            pl.BlockSpec((M, C * V, B), lambda i: (0, 0, i)),
            pl.BlockSpec((1, 1), lambda i: (0, 0)),
        ],
        out_specs=pl.BlockSpec((M, C * V, B), lambda i: (0, 0, i)),
        out_shape=jax.ShapeDtypeStruct((M, C * V, NT), jnp.bfloat16),
    )(xt, lam2)
    out = out3.reshape(M, C, V, N, T).transpose(3, 1, 4, 2, 0)
    return out.astype(jnp.float32)


# speedup vs baseline: 1.0028x; 1.0028x over previous
import jax
import jax.numpy as jnp
from jax.experimental import pallas as pl

K = 2
TAU = 0.35
ALPHA_SELFLOOP = 0.5
EPS = 1e-06
LAMBDA_FUSE = 0.05

_M = 10
_CV = 100


def _taglayer_body(x_ref, lam_ref, o_ref):
    B = x_ref.shape[-1]
    f32 = jnp.float32

    cen = []
    mag = []
    cen_d = [[], [], []]
    for m in range(_M):
        g = x_ref[m][0:75, :].astype(f32)
        mag.append(jnp.sum(jnp.abs(g), axis=0))
        for d in range(3):
            cen_d[d].append(jnp.mean(g[d * 25:(d + 1) * 25, :], axis=0))
    presf = (jnp.stack(mag, axis=0) > EPS).astype(f32)

    dsq = jnp.zeros((_M, _M, B), dtype=f32)
    for d in range(3):
        cd = jnp.stack(cen_d[d], axis=0)
        diff = cd[:, None, :] - cd[None, :, :]
        dsq = dsq + diff * diff
    dist = jnp.sqrt(jnp.clip(dsq, 1e-12, None))
    pair_ok = presf[:, None, :] * presf[None, :, :] > 0
    dist = jnp.where(pair_ok, dist, 1000000.0)
    ii = jax.lax.broadcasted_iota(jnp.int32, (_M, _M, B), 0)
    jj = jax.lax.broadcasted_iota(jnp.int32, (_M, _M, B), 1)
    eye3 = (ii == jj).astype(f32)
    dist = dist + eye3 * 1000000.0

    d1 = dist[0]
    i1 = jnp.zeros((_M, B), dtype=jnp.int32)
    for j in range(1, _M):
        dj = dist[j]
        take = dj < d1
        d1 = jnp.where(take, dj, d1)
        i1 = jnp.where(take, j, i1)
    big = jnp.float32(3.0e38)
    d2 = jnp.where(i1 == 0, big, dist[0])
    i2 = jnp.zeros((_M, B), dtype=jnp.int32)
    for j in range(1, _M):
        dj = dist[j]
        take = jnp.logical_and(i1 != j, dj < d2)
        d2 = jnp.where(take, dj, d2)
        i2 = jnp.where(take, j, i2)

    w1 = jnp.exp(-d1 / TAU)
    w2 = jnp.exp(-d2 / TAU)
    s = w1 + w2 + 1e-06
    w1 = w1 / s
    w2 = w2 / s
    cols = []
    for j in range(_M):
        cols.append(w1 * (i1 == j).astype(f32) + w2 * (i2 == j).astype(f32))
    adjT = jnp.stack(cols, axis=0)

    adjT = 0.5 * (adjT + jnp.transpose(adjT, (1, 0, 2)))
    adjT = adjT + ALPHA_SELFLOOP * eye3
    rs = jnp.sum(adjT, axis=0)
    adjT = adjT / (rs + 1e-06)[None, :, :]
    deg = jnp.clip(jnp.sum(adjT, axis=0), 1e-06, None)
    dinv = jax.lax.rsqrt(deg)
    dinv_j = dinv[:, None, :]
    adjT = adjT * dinv[None, :, :] * dinv_j

    lamb = lam_ref[0, 0].astype(jnp.bfloat16)
    adjb = adjT.astype(jnp.bfloat16)
    for i in range(_M):
        acc = None
        for j in range(_M):
            t = adjb[j, i:i + 1, :] * x_ref[j]
            acc = t if acc is None else acc + t
        o_ref[i] = x_ref[i] + lamb * acc


def kernel(x, lam):
    N, C, T, V, M = x.shape
    NT = N * T
    B = 384
    xt = jnp.transpose(x.astype(jnp.bfloat16), (4, 1, 3, 0, 2))
    xt = xt.reshape(M, C * V, NT)
    lam2 = jnp.asarray(lam, jnp.float32).reshape(1, 1)
    out3 = pl.pallas_call(
        _taglayer_body,
        grid=(NT // B,),
        in_specs=[
            pl.BlockSpec((M, C * V, B), lambda i: (0, 0, i)),
            pl.BlockSpec((1, 1), lambda i: (0, 0)),
        ],
        out_specs=pl.BlockSpec((M, C * V, B), lambda i: (0, 0, i)),
        out_shape=jax.ShapeDtypeStruct((M, C * V, NT), jnp.bfloat16),
    )(xt, lam2)
    out = out3.reshape(M, C, V, N, T).transpose(3, 1, 4, 2, 0)
    return out.astype(jnp.float32)
